# Initial kernel scaffold; baseline (speedup 1.0000x reference)
import functools

import jax
import jax.numpy as jnp
from jax import lax
from jax.experimental import pallas as pl
from jax.experimental.pallas import tpu as pltpu
from jax.experimental.pallas import tpu_sc as plsc

N_NODES = 10000
N_EDGES = 320000
D_IN = 128
D_HID = 64
D_OUT = 128

NC = 2
NS = 16
NW = NC * NS
CHUNK = 128
CHUNKS = 79
E_PAD = NW * CHUNKS * CHUNK
N_PAD = CHUNKS * CHUNK
ROWS_PER_TILE = N_PAD // NS
ROW_BLOCK = 1264
N_ROW_BLOCKS = N_PAD // ROW_BLOCK


def _mesh():
    return plsc.VectorSubcoreMesh(core_axis_name="c", subcore_axis_name="s")


@functools.partial(
    pl.kernel,
    out_type=jax.ShapeDtypeStruct((NC, N_PAD), jnp.float32),
    mesh=_mesh(),
    scratch_types=[
        pltpu.VMEM((CHUNKS, CHUNK), jnp.int32),
        pltpu.VMEM((CHUNK,), jnp.float32),
        pltpu.VMEM_SHARED((N_PAD,), jnp.float32),
    ],
)
def _count_kernel(dst_hbm, zeros_hbm, cnt_hbm, idx_v, ones_v, acc_s):
    c = lax.axis_index("c")
    s = lax.axis_index("s")
    wid = c * NS + s
    pltpu.sync_copy(dst_hbm.at[wid], idx_v)
    for i in range(CHUNK // 16):
        ones_v[pl.ds(i * 16, 16)] = jnp.ones((16,), jnp.float32)
    tile_rows = pl.ds(s * ROWS_PER_TILE, ROWS_PER_TILE)
    pltpu.sync_copy(zeros_hbm, acc_s.at[tile_rows])
    plsc.subcore_barrier()

    def chunk_body(j, carry):
        pltpu.sync_copy(ones_v, acc_s.at[idx_v.at[j]], add=True)
        return carry

    lax.fori_loop(0, CHUNKS, chunk_body, 0)
    plsc.subcore_barrier()
    pltpu.sync_copy(acc_s.at[tile_rows], cnt_hbm.at[c, tile_rows])


def _make_scatter(D):
    @functools.partial(
        pl.kernel,
        out_type=jax.ShapeDtypeStruct((NC, N_PAD, D), jnp.float32),
        mesh=_mesh(),
        scratch_types=[
            pltpu.VMEM((CHUNKS, CHUNK), jnp.int32),
            pltpu.VMEM((CHUNKS, CHUNK), jnp.int32),
            pltpu.VMEM((CHUNK, D), jnp.float32),
            pltpu.VMEM((CHUNK, D), jnp.float32),
            pltpu.VMEM_SHARED((N_PAD, D), jnp.float32),
            pltpu.SemaphoreType.DMA,
            pltpu.SemaphoreType.DMA,
        ],
    )
    def _scatter(src_hbm, dst_hbm, xh_hbm, zeros_hbm, out_hbm,
                 srcv, dstv, rows0, rows1, acc_s, sem0, sem1):
        c = lax.axis_index("c")
        s = lax.axis_index("s")
        wid = c * NS + s
        pltpu.sync_copy(src_hbm.at[wid], srcv)
        pltpu.sync_copy(dst_hbm.at[wid], dstv)
        tile_rows = pl.ds(s * ROWS_PER_TILE, ROWS_PER_TILE)
        pltpu.sync_copy(zeros_hbm, acc_s.at[tile_rows])
        plsc.subcore_barrier()

        pltpu.async_copy(xh_hbm.at[srcv.at[0]], rows0, sem0)
        pltpu.async_copy(xh_hbm.at[srcv.at[1]], rows1, sem1)

        def pair_body(p, carry):
            j0 = p * 2
            pltpu.make_async_copy(xh_hbm.at[srcv.at[j0]], rows0, sem0).wait()
            pltpu.sync_copy(rows0, acc_s.at[dstv.at[j0]], add=True)
            pltpu.async_copy(xh_hbm.at[srcv.at[j0 + 2]], rows0, sem0)
            pltpu.make_async_copy(
                xh_hbm.at[srcv.at[j0 + 1]], rows1, sem1).wait()
            pltpu.sync_copy(rows1, acc_s.at[dstv.at[j0 + 1]], add=True)

            @pl.when(j0 + 3 < CHUNKS)
            def _():
                pltpu.async_copy(xh_hbm.at[srcv.at[j0 + 3]], rows1, sem1)

            return carry

        lax.fori_loop(0, CHUNKS // 2, pair_body, 0)
        jl = CHUNKS - 1
        pltpu.make_async_copy(xh_hbm.at[srcv.at[jl]], rows0, sem0).wait()
        pltpu.sync_copy(rows0, acc_s.at[dstv.at[jl]], add=True)

        plsc.subcore_barrier()
        pltpu.sync_copy(acc_s.at[tile_rows], out_hbm.at[c, tile_rows])

    return _scatter


_scatter_hid = _make_scatter(D_HID)
_scatter_out = _make_scatter(D_OUT)


def _tc1_body(cnt_ref, x_ref, w_ref, xh_ref, dinv_ref):
    cnt = cnt_ref[0] + cnt_ref[1]
    dinv = lax.rsqrt(cnt + 1.0)
    h = jnp.dot(x_ref[...], w_ref[...], preferred_element_type=jnp.float32)
    xh_ref[...] = h * dinv
    dinv_ref[...] = dinv


def _tc1(cnt2, x_p, W1):
    return pl.pallas_call(
        _tc1_body,
        grid=(N_ROW_BLOCKS,),
        in_specs=[
            pl.BlockSpec((NC, ROW_BLOCK, 1), lambda i: (0, i, 0)),
            pl.BlockSpec((ROW_BLOCK, D_IN), lambda i: (i, 0)),
            pl.BlockSpec((D_IN, D_HID), lambda i: (0, 0)),
        ],
        out_specs=[
            pl.BlockSpec((ROW_BLOCK, D_HID), lambda i: (i, 0)),
            pl.BlockSpec((ROW_BLOCK, 1), lambda i: (i, 0)),
        ],
        out_shape=[
            jax.ShapeDtypeStruct((N_PAD, D_HID), jnp.float32),
            jax.ShapeDtypeStruct((N_PAD, 1), jnp.float32),
        ],
    )(cnt2, x_p, W1)


def _tc2_body(acc_ref, xh_ref, dinv_ref, b_ref, w_ref, out_ref):
    dinv = dinv_ref[...]
    z = (acc_ref[0] + acc_ref[1] + xh_ref[...]) * dinv + b_ref[...]
    a = jnp.maximum(z, 0.0)
    out_ref[...] = jnp.dot(
        a, w_ref[...], preferred_element_type=jnp.float32) * dinv


def _tc2(acc1, xh1, dinv, b1, W2):
    return pl.pallas_call(
        _tc2_body,
        grid=(N_ROW_BLOCKS,),
        in_specs=[
            pl.BlockSpec((NC, ROW_BLOCK, D_HID), lambda i: (0, i, 0)),
            pl.BlockSpec((ROW_BLOCK, D_HID), lambda i: (i, 0)),
            pl.BlockSpec((ROW_BLOCK, 1), lambda i: (i, 0)),
            pl.BlockSpec((1, D_HID), lambda i: (0, 0)),
            pl.BlockSpec((D_HID, D_OUT), lambda i: (0, 0)),
        ],
        out_specs=pl.BlockSpec((ROW_BLOCK, D_OUT), lambda i: (i, 0)),
        out_shape=jax.ShapeDtypeStruct((N_PAD, D_OUT), jnp.float32),
    )(acc1, xh1, dinv, b1, W2)


def _tc3_body(acc_ref, xh_ref, dinv_ref, b_ref, out_ref):
    out_ref[...] = (acc_ref[0] + acc_ref[1] + xh_ref[...]) * dinv_ref[...] \
        + b_ref[...]


def _tc3(acc2, xh2, dinv, b2):
    return pl.pallas_call(
        _tc3_body,
        grid=(N_ROW_BLOCKS,),
        in_specs=[
            pl.BlockSpec((NC, ROW_BLOCK, D_OUT), lambda i: (0, i, 0)),
            pl.BlockSpec((ROW_BLOCK, D_OUT), lambda i: (i, 0)),
            pl.BlockSpec((ROW_BLOCK, 1), lambda i: (i, 0)),
            pl.BlockSpec((1, D_OUT), lambda i: (0, 0)),
        ],
        out_specs=pl.BlockSpec((ROW_BLOCK, D_OUT), lambda i: (i, 0)),
        out_shape=jax.ShapeDtypeStruct((N_PAD, D_OUT), jnp.float32),
    )(acc2, xh2, dinv, b2)


def kernel(x, edge_index, W1, b1, W2, b2):
    src = edge_index[0].astype(jnp.int32)
    dst = edge_index[1].astype(jnp.int32)
    npad_e = E_PAD - N_EDGES
    pad_src = jnp.full((npad_e,), N_NODES, jnp.int32)
    pad_dst = N_NODES + (jnp.arange(npad_e, dtype=jnp.int32)
                         % (N_PAD - N_NODES))
    src3 = jnp.concatenate([src, pad_src]).reshape(NW, CHUNKS, CHUNK)
    dst3 = jnp.concatenate([dst, pad_dst]).reshape(NW, CHUNKS, CHUNK)
    x_p = jnp.zeros((N_PAD, D_IN), jnp.float32).at[:N_NODES].set(x)

    zeros1 = jnp.zeros((ROWS_PER_TILE,), jnp.float32)
    zeros_hid = jnp.zeros((ROWS_PER_TILE, D_HID), jnp.float32)
    zeros_out = jnp.zeros((ROWS_PER_TILE, D_OUT), jnp.float32)

    cnt2 = _count_kernel(dst3, zeros1)
    xh1, dinv = _tc1(cnt2.reshape(NC, N_PAD, 1), x_p, W1)
    acc1 = _scatter_hid(src3, dst3, xh1, zeros_hid)
    xh2 = _tc2(acc1, xh1, dinv, b1.reshape(1, D_HID), W2)
    acc2 = _scatter_out(src3, dst3, xh2, zeros_out)
    out = _tc3(acc2, xh2, dinv, b2.reshape(1, D_OUT))
    return out[:N_NODES]


# speedup vs baseline: 20.5431x; 20.5431x over previous
import functools

import jax
import jax.numpy as jnp
from jax import lax
from jax.experimental import pallas as pl
from jax.experimental.pallas import tpu as pltpu
from jax.experimental.pallas import tpu_sc as plsc

N_NODES = 10000
N_EDGES = 320000
D_IN = 128
D_HID = 64
D_OUT = 128

NC = 2
NS = 16
NW = NC * NS
CHUNK = 128
CHUNKS = 79
E_PAD = NW * CHUNKS * CHUNK
EDGES_PER_TILE = E_PAD // NW
N_PAD = 10240
ROWS_PER_TILE = N_PAD // NS
ZCHUNKS = ROWS_PER_TILE // CHUNK
ROW_BLOCK = 1280
N_ROW_BLOCKS = N_PAD // ROW_BLOCK


def _mesh():
    return plsc.VectorSubcoreMesh(core_axis_name="c", subcore_axis_name="s",
                                  num_cores=NC, num_subcores=NS)


@functools.partial(
    pl.kernel,
    out_type=jax.ShapeDtypeStruct((NC, N_PAD), jnp.float32),
    mesh=_mesh(),
    scratch_types=[
        pltpu.VMEM((CHUNKS, CHUNK), jnp.int32),
        pltpu.VMEM((CHUNK,), jnp.float32),
        pltpu.VMEM((ROWS_PER_TILE,), jnp.float32),
        pltpu.VMEM_SHARED((N_PAD,), jnp.float32),
    ],
    compiler_params=pltpu.CompilerParams(use_tc_tiling_on_sc=False),
)
def _count_kernel(dst_hbm, cnt_hbm, idx_v, ones_v, tmp_v, acc_s):
    c = lax.axis_index("c")
    s = lax.axis_index("s")
    wid = c * NS + s
    pltpu.sync_copy(dst_hbm.at[wid], idx_v)
    for i in range(CHUNK // 16):
        ones_v[pl.ds(i * 16, 16)] = jnp.ones((16,), jnp.float32)

    def zero_body(i, carry):
        tmp_v[pl.ds(i * 16, 16)] = jnp.zeros((16,), jnp.float32)
        return carry

    lax.fori_loop(0, ROWS_PER_TILE // 16, zero_body, 0)
    tile_rows = pl.ds(s * ROWS_PER_TILE, ROWS_PER_TILE)
    pltpu.sync_copy(tmp_v, acc_s.at[tile_rows])
    plsc.subcore_barrier()

    def chunk_body(j, carry):
        pltpu.sync_copy(ones_v, acc_s.at[idx_v.at[j]], add=True)
        return carry

    lax.fori_loop(0, CHUNKS, chunk_body, 0)
    plsc.subcore_barrier()
    pltpu.sync_copy(acc_s.at[tile_rows], tmp_v)
    pltpu.sync_copy(tmp_v, cnt_hbm.at[c, tile_rows])


def _make_scatter(D, chunk):
    chunks = EDGES_PER_TILE // chunk
    zchunks = ROWS_PER_TILE // chunk
    @functools.partial(
        pl.kernel,
        out_type=jax.ShapeDtypeStruct((NC, N_PAD, D), jnp.float32),
        mesh=_mesh(),
        scratch_types=[
            pltpu.VMEM((chunks, chunk), jnp.int32),
            pltpu.VMEM((chunks, chunk), jnp.int32),
            pltpu.VMEM((chunk, D), jnp.float32),
            pltpu.VMEM((chunk, D), jnp.float32),
            pltpu.VMEM_SHARED((N_PAD, D), jnp.float32),
            pltpu.SemaphoreType.DMA,
            pltpu.SemaphoreType.DMA,
        ],
        compiler_params=pltpu.CompilerParams(use_tc_tiling_on_sc=False),
    )
    def _scatter(src_hbm, dst_hbm, xh_hbm, out_hbm,
                 srcv, dstv, rows0, rows1, acc_s, sem0, sem1):
        c = lax.axis_index("c")
        s = lax.axis_index("s")
        wid = c * NS + s
        pltpu.sync_copy(src_hbm.at[wid], srcv)
        pltpu.sync_copy(dst_hbm.at[wid], dstv)

        def zero_body(r, carry):
            for i in range(D // 16):
                rows0[r, pl.ds(i * 16, 16)] = jnp.zeros((16,), jnp.float32)
            return carry

        lax.fori_loop(0, chunk, zero_body, 0)
        for z in range(zchunks):
            pltpu.sync_copy(
                rows0, acc_s.at[pl.ds(s * ROWS_PER_TILE + z * chunk, chunk)])
        plsc.subcore_barrier()

        pltpu.async_copy(xh_hbm.at[srcv.at[0]], rows0, sem0)
        pltpu.async_copy(xh_hbm.at[srcv.at[1]], rows1, sem1)

        def pair_body(p, carry):
            j0 = p * 2
            pltpu.make_async_copy(xh_hbm.at[srcv.at[j0]], rows0, sem0).wait()
            pltpu.sync_copy(rows0, acc_s.at[dstv.at[j0]], add=True)

            @pl.when(j0 + 2 < chunks)
            def _():
                pltpu.async_copy(xh_hbm.at[srcv.at[j0 + 2]], rows0, sem0)

            pltpu.make_async_copy(
                xh_hbm.at[srcv.at[j0 + 1]], rows1, sem1).wait()
            pltpu.sync_copy(rows1, acc_s.at[dstv.at[j0 + 1]], add=True)

            @pl.when(j0 + 3 < chunks)
            def _():
                pltpu.async_copy(xh_hbm.at[srcv.at[j0 + 3]], rows1, sem1)

            return carry

        lax.fori_loop(0, chunks // 2, pair_body, 0)
        if chunks % 2:
            jl = chunks - 1
            pltpu.make_async_copy(xh_hbm.at[srcv.at[jl]], rows0, sem0).wait()
            pltpu.sync_copy(rows0, acc_s.at[dstv.at[jl]], add=True)

        plsc.subcore_barrier()
        for z in range(zchunks):
            buf = rows0 if z % 2 == 0 else rows1
            rows_z = pl.ds(s * ROWS_PER_TILE + z * chunk, chunk)
            pltpu.sync_copy(acc_s.at[rows_z], buf)
            pltpu.sync_copy(buf, out_hbm.at[c, rows_z])

    return _scatter


_scatter_hid = _make_scatter(D_HID, 128)
_scatter_out = _make_scatter(D_OUT, 64)


def _tc1_body(cnt_ref, x_ref, w_ref, xh_ref, dinv_ref):
    cnt = cnt_ref[0] + cnt_ref[1]
    dinv = lax.rsqrt(cnt + 1.0)
    h = jnp.dot(x_ref[...], w_ref[...], preferred_element_type=jnp.float32)
    xh_ref[...] = h * dinv
    dinv_ref[...] = dinv


def _tc1(cnt2, x_p, W1):
    return pl.pallas_call(
        _tc1_body,
        grid=(N_ROW_BLOCKS,),
        in_specs=[
            pl.BlockSpec((NC, ROW_BLOCK, 1), lambda i: (0, i, 0)),
            pl.BlockSpec((ROW_BLOCK, D_IN), lambda i: (i, 0)),
            pl.BlockSpec((D_IN, D_HID), lambda i: (0, 0)),
        ],
        out_specs=[
            pl.BlockSpec((ROW_BLOCK, D_HID), lambda i: (i, 0)),
            pl.BlockSpec((ROW_BLOCK, 1), lambda i: (i, 0)),
        ],
        out_shape=[
            jax.ShapeDtypeStruct((N_PAD, D_HID), jnp.float32),
            jax.ShapeDtypeStruct((N_PAD, 1), jnp.float32),
        ],
    )(cnt2, x_p, W1)


def _tc2_body(acc_ref, xh_ref, dinv_ref, b_ref, w_ref, out_ref):
    dinv = dinv_ref[...]
    z = (acc_ref[0] + acc_ref[1] + xh_ref[...]) * dinv + b_ref[...]
    a = jnp.maximum(z, 0.0)
    out_ref[...] = jnp.dot(
        a, w_ref[...], preferred_element_type=jnp.float32) * dinv


def _tc2(acc1, xh1, dinv, b1, W2):
    return pl.pallas_call(
        _tc2_body,
        grid=(N_ROW_BLOCKS,),
        in_specs=[
            pl.BlockSpec((NC, ROW_BLOCK, D_HID), lambda i: (0, i, 0)),
            pl.BlockSpec((ROW_BLOCK, D_HID), lambda i: (i, 0)),
            pl.BlockSpec((ROW_BLOCK, 1), lambda i: (i, 0)),
            pl.BlockSpec((1, D_HID), lambda i: (0, 0)),
            pl.BlockSpec((D_HID, D_OUT), lambda i: (0, 0)),
        ],
        out_specs=pl.BlockSpec((ROW_BLOCK, D_OUT), lambda i: (i, 0)),
        out_shape=jax.ShapeDtypeStruct((N_PAD, D_OUT), jnp.float32),
    )(acc1, xh1, dinv, b1, W2)


def _tc3_body(acc_ref, xh_ref, dinv_ref, b_ref, out_ref):
    out_ref[...] = (acc_ref[0] + acc_ref[1] + xh_ref[...]) * dinv_ref[...] \
        + b_ref[...]


def _tc3(acc2, xh2, dinv, b2):
    return pl.pallas_call(
        _tc3_body,
        grid=(N_ROW_BLOCKS,),
        in_specs=[
            pl.BlockSpec((NC, ROW_BLOCK, D_OUT), lambda i: (0, i, 0)),
            pl.BlockSpec((ROW_BLOCK, D_OUT), lambda i: (i, 0)),
            pl.BlockSpec((ROW_BLOCK, 1), lambda i: (i, 0)),
            pl.BlockSpec((1, D_OUT), lambda i: (0, 0)),
        ],
        out_specs=pl.BlockSpec((ROW_BLOCK, D_OUT), lambda i: (i, 0)),
        out_shape=jax.ShapeDtypeStruct((N_PAD, D_OUT), jnp.float32),
    )(acc2, xh2, dinv, b2)


def kernel(x, edge_index, W1, b1, W2, b2):
    src = edge_index[0].astype(jnp.int32)
    dst = edge_index[1].astype(jnp.int32)
    npad_e = E_PAD - N_EDGES
    pad_src = jnp.full((npad_e,), N_NODES, jnp.int32)
    pad_dst = N_NODES + (jnp.arange(npad_e, dtype=jnp.int32)
                         % (N_PAD - N_NODES))
    src_p = jnp.concatenate([src, pad_src])
    dst_p = jnp.concatenate([dst, pad_dst])
    src3 = src_p.reshape(NW, CHUNKS, CHUNK)
    dst3 = dst_p.reshape(NW, CHUNKS, CHUNK)
    src3b = src_p.reshape(NW, EDGES_PER_TILE // 64, 64)
    dst3b = dst_p.reshape(NW, EDGES_PER_TILE // 64, 64)
    x_p = jnp.zeros((N_PAD, D_IN), jnp.float32).at[:N_NODES].set(x)

    cnt2 = _count_kernel(dst3)
    xh1, dinv = _tc1(cnt2.reshape(NC, N_PAD, 1), x_p, W1)
    acc1 = _scatter_hid(src3, dst3, xh1)
    xh2 = _tc2(acc1, xh1, dinv, b1.reshape(1, D_HID), W2)
    acc2 = _scatter_out(src3b, dst3b, xh2)
    out = _tc3(acc2, xh2, dinv, b2.reshape(1, D_OUT))
    return out[:N_NODES]
